# baseline (speedup 1.0000x reference)
import functools

import jax
import jax.numpy as jnp
from jax import lax
from jax.experimental import pallas as pl
from jax.experimental.pallas import tpu as pltpu
from jax.experimental.pallas import tpu_sc as plsc

_NUM_OBJS = 1000
_NUM_RELS = 64
_BATCH = 16384
_R = _NUM_OBJS * _NUM_OBJS

_INFO = plsc.get_sparse_core_info()
_NC = _INFO.num_cores
_NS = _INFO.num_subcores
_NW = _NC * _NS
_L = _INFO.num_lanes

_BPW = _BATCH // _NW
_CHUNK = 128
_NCHUNK = _BPW // _CHUNK
_W2 = 2 * _NUM_RELS

_BLK = 4096
_SH = _BLK.bit_length() - 1
_NBLK = -(-_R // _BLK)
_NPAIR = -(-_R // (2 * _BLK))
_RP = _NPAIR * _BLK


def _repack_block(tl_ref, tr_ref, out_ref):
    eye = jnp.eye(_NUM_RELS, dtype=jnp.float32)

    def xpose(x):
        return lax.dot_general(
            x, eye, (((0,), (0,)), ((), ())),
            precision=lax.Precision.HIGHEST,
            preferred_element_type=jnp.float32)

    out_ref[...] = jnp.concatenate(
        [xpose(tl_ref[...]), xpose(tr_ref[...])], axis=1)


@jax.jit
def _repack(table_t):
    return pl.pallas_call(
        _repack_block,
        grid=(_NPAIR,),
        in_specs=[
            pl.BlockSpec((_NUM_RELS, _BLK), lambda k: (0, 2 * k)),
            pl.BlockSpec(
                (_NUM_RELS, _BLK),
                lambda k: (0, jnp.minimum(2 * k + 1, _NBLK - 1))),
        ],
        out_specs=pl.BlockSpec((_BLK, _W2), lambda k: (k, 0)),
        out_shape=jax.ShapeDtypeStruct((_RP, _W2), jnp.float32),
    )(table_t, table_t)


def _make_gather():
    mesh = plsc.VectorSubcoreMesh(core_axis_name="c", subcore_axis_name="s")

    @functools.partial(
        pl.kernel,
        mesh=mesh,
        compiler_params=pltpu.CompilerParams(use_tc_tiling_on_sc=False),
        out_type=jax.ShapeDtypeStruct((_BATCH, _NUM_RELS), jnp.float32),
        scratch_types=[
            pltpu.VMEM((_BPW,), jnp.int32),
            pltpu.VMEM((_BPW,), jnp.int32),
            pltpu.VMEM((_NCHUNK, _CHUNK), jnp.int32),
            pltpu.VMEM((_BPW, _NUM_RELS), jnp.float32),
            pltpu.SemaphoreType.DMA,
        ],
    )
    def gather_kernel(l0_hbm, l1_hbm, table_hbm, out_hbm,
                      l0_v, l1_v, idx_v, rows_v, sem):
        wid = lax.axis_index("s") * _NC + lax.axis_index("c")
        base = wid * _BPW

        pltpu.sync_copy(l0_hbm.at[pl.ds(base, _BPW)], l0_v)
        pltpu.sync_copy(l1_hbm.at[pl.ds(base, _BPW)], l1_v)

        copies = []
        for c in range(_NCHUNK):
            for k in range(_CHUNK // _L):
                s = pl.ds((c * (_CHUNK // _L) + k) * _L, _L)
                idx = l0_v[s] * _NUM_OBJS + l1_v[s]
                idx_v[c, pl.ds(k * _L, _L)] = (
                    lax.shift_left(lax.shift_right_logical(idx, _SH + 1),
                                   _SH + 1)
                    + lax.shift_left(idx & (_BLK - 1), 1)
                    + (lax.shift_right_logical(idx, _SH) & 1))
            copies.append(
                pltpu.async_copy(
                    table_hbm.at[idx_v.at[c]],
                    rows_v.at[pl.ds(c * _CHUNK, _CHUNK)],
                    sem,
                )
            )
        for cp in copies:
            cp.wait()

        pltpu.sync_copy(rows_v, out_hbm.at[pl.ds(base, _BPW)])

    return gather_kernel


_GATHER = _make_gather()


@jax.jit
def kernel(labels, obj_baseline):
    table2 = _repack(obj_baseline.T)
    table64 = table2.reshape(2 * _RP, _NUM_RELS)
    return _GATHER(labels[:, 0], labels[:, 1], table64)


# speedup vs baseline: 2.6751x; 2.6751x over previous
import functools

import jax
import jax.numpy as jnp
from jax import lax
from jax.experimental import pallas as pl
from jax.experimental.pallas import tpu as pltpu
from jax.experimental.pallas import tpu_sc as plsc

_NUM_OBJS = 1000
_NUM_RELS = 64
_BATCH = 16384
_R = _NUM_OBJS * _NUM_OBJS

_INFO = plsc.get_sparse_core_info()
_NC = _INFO.num_cores
_NS = _INFO.num_subcores
_NW = _NC * _NS
_L = _INFO.num_lanes

_BPW = _BATCH // _NW
_CHUNK = 128
_NCHUNK = _BPW // _CHUNK
_W2 = 2 * _NUM_RELS

_BLK = 8192
_SH = _BLK.bit_length() - 1
_NBLK = -(-_R // _BLK)
_NPAIR = -(-_R // (2 * _BLK))
_RP = _NPAIR * _BLK


def _repack_block(tl_ref, tr_ref, out_ref):
    xcat = jnp.concatenate([tl_ref[...], tr_ref[...]], axis=0)
    eye = jnp.eye(_W2, dtype=jnp.float32)
    out_ref[...] = lax.dot_general(
        xcat, eye, (((0,), (0,)), ((), ())),
        preferred_element_type=jnp.float32)


@jax.jit
def _repack(table_t):
    return pl.pallas_call(
        _repack_block,
        grid=(_NPAIR,),
        in_specs=[
            pl.BlockSpec((_NUM_RELS, _BLK), lambda k: (0, 2 * k)),
            pl.BlockSpec(
                (_NUM_RELS, _BLK),
                lambda k: (0, jnp.minimum(2 * k + 1, _NBLK - 1))),
        ],
        out_specs=pl.BlockSpec((_BLK, _W2), lambda k: (k, 0)),
        out_shape=jax.ShapeDtypeStruct((_RP, _W2), jnp.float32),
    )(table_t, table_t)


def _make_gather():
    mesh = plsc.VectorSubcoreMesh(core_axis_name="c", subcore_axis_name="s")

    @functools.partial(
        pl.kernel,
        mesh=mesh,
        compiler_params=pltpu.CompilerParams(use_tc_tiling_on_sc=False),
        out_type=jax.ShapeDtypeStruct((_BATCH, _NUM_RELS), jnp.float32),
        scratch_types=[
            pltpu.VMEM((_BPW,), jnp.int32),
            pltpu.VMEM((_BPW,), jnp.int32),
            pltpu.VMEM((_NCHUNK, _CHUNK), jnp.int32),
            pltpu.VMEM((_BPW, _NUM_RELS), jnp.float32),
            pltpu.SemaphoreType.DMA,
        ],
    )
    def gather_kernel(l0_hbm, l1_hbm, table_hbm, out_hbm,
                      l0_v, l1_v, idx_v, rows_v, sem):
        wid = lax.axis_index("s") * _NC + lax.axis_index("c")
        base = wid * _BPW

        pltpu.sync_copy(l0_hbm.at[pl.ds(base, _BPW)], l0_v)
        pltpu.sync_copy(l1_hbm.at[pl.ds(base, _BPW)], l1_v)

        copies = []
        for c in range(_NCHUNK):
            for k in range(_CHUNK // _L):
                s = pl.ds((c * (_CHUNK // _L) + k) * _L, _L)
                idx = l0_v[s] * _NUM_OBJS + l1_v[s]
                idx_v[c, pl.ds(k * _L, _L)] = (
                    lax.shift_left(lax.shift_right_logical(idx, _SH + 1),
                                   _SH + 1)
                    + lax.shift_left(idx & (_BLK - 1), 1)
                    + (lax.shift_right_logical(idx, _SH) & 1))
            copies.append(
                pltpu.async_copy(
                    table_hbm.at[idx_v.at[c]],
                    rows_v.at[pl.ds(c * _CHUNK, _CHUNK)],
                    sem,
                )
            )
        for cp in copies:
            cp.wait()

        pltpu.sync_copy(rows_v, out_hbm.at[pl.ds(base, _BPW)])

    return gather_kernel


_GATHER = _make_gather()


@jax.jit
def kernel(labels, obj_baseline):
    table2 = _repack(obj_baseline.T)
    table64 = table2.reshape(2 * _RP, _NUM_RELS)
    return _GATHER(labels[:, 0], labels[:, 1], table64)


# speedup vs baseline: 2.7304x; 1.0207x over previous
import functools

import jax
import jax.numpy as jnp
from jax import lax
from jax.experimental import pallas as pl
from jax.experimental.pallas import tpu as pltpu
from jax.experimental.pallas import tpu_sc as plsc

_NUM_OBJS = 1000
_NUM_RELS = 64
_BATCH = 16384
_R = _NUM_OBJS * _NUM_OBJS

_INFO = plsc.get_sparse_core_info()
_NC = _INFO.num_cores
_NS = _INFO.num_subcores
_NW = _NC * _NS
_L = _INFO.num_lanes

_BPW = _BATCH // _NW
_CHUNK = 128
_NCHUNK = _BPW // _CHUNK
_W2 = 2 * _NUM_RELS

_BLK = 16384
_SH = _BLK.bit_length() - 1
_NBLK = -(-_R // _BLK)
_NPAIR = -(-_R // (2 * _BLK))
_RP = _NPAIR * _BLK


def _repack_block(tl_ref, tr_ref, out_ref):
    xcat = jnp.concatenate([tl_ref[...], tr_ref[...]], axis=0)
    eye = jnp.eye(_W2, dtype=jnp.float32)
    out_ref[...] = lax.dot_general(
        xcat, eye, (((0,), (0,)), ((), ())),
        preferred_element_type=jnp.float32)


@jax.jit
def _repack(table_t):
    return pl.pallas_call(
        _repack_block,
        grid=(_NPAIR,),
        in_specs=[
            pl.BlockSpec((_NUM_RELS, _BLK), lambda k: (0, 2 * k)),
            pl.BlockSpec(
                (_NUM_RELS, _BLK),
                lambda k: (0, jnp.minimum(2 * k + 1, _NBLK - 1))),
        ],
        out_specs=pl.BlockSpec((_BLK, _W2), lambda k: (k, 0)),
        out_shape=jax.ShapeDtypeStruct((_RP, _W2), jnp.float32),
    )(table_t, table_t)


def _make_gather():
    mesh = plsc.VectorSubcoreMesh(core_axis_name="c", subcore_axis_name="s")

    @functools.partial(
        pl.kernel,
        mesh=mesh,
        compiler_params=pltpu.CompilerParams(use_tc_tiling_on_sc=False),
        out_type=jax.ShapeDtypeStruct((_BATCH, _NUM_RELS), jnp.float32),
        scratch_types=[
            pltpu.VMEM((_BPW,), jnp.int32),
            pltpu.VMEM((_BPW,), jnp.int32),
            pltpu.VMEM((_NCHUNK, _CHUNK), jnp.int32),
            pltpu.VMEM((_BPW, _NUM_RELS), jnp.float32),
            pltpu.SemaphoreType.DMA,
        ],
    )
    def gather_kernel(l0_hbm, l1_hbm, table_hbm, out_hbm,
                      l0_v, l1_v, idx_v, rows_v, sem):
        wid = lax.axis_index("s") * _NC + lax.axis_index("c")
        base = wid * _BPW

        pltpu.sync_copy(l0_hbm.at[pl.ds(base, _BPW)], l0_v)
        pltpu.sync_copy(l1_hbm.at[pl.ds(base, _BPW)], l1_v)

        copies = []
        for c in range(_NCHUNK):
            for k in range(_CHUNK // _L):
                s = pl.ds((c * (_CHUNK // _L) + k) * _L, _L)
                idx = l0_v[s] * _NUM_OBJS + l1_v[s]
                idx_v[c, pl.ds(k * _L, _L)] = (
                    lax.shift_left(lax.shift_right_logical(idx, _SH + 1),
                                   _SH + 1)
                    + lax.shift_left(idx & (_BLK - 1), 1)
                    + (lax.shift_right_logical(idx, _SH) & 1))
            copies.append(
                pltpu.async_copy(
                    table_hbm.at[idx_v.at[c]],
                    rows_v.at[pl.ds(c * _CHUNK, _CHUNK)],
                    sem,
                )
            )
        for cp in copies:
            cp.wait()

        pltpu.sync_copy(rows_v, out_hbm.at[pl.ds(base, _BPW)])

    return gather_kernel


_GATHER = _make_gather()


@jax.jit
def kernel(labels, obj_baseline):
    table2 = _repack(obj_baseline.T)
    table64 = table2.reshape(2 * _RP, _NUM_RELS)
    return _GATHER(labels[:, 0], labels[:, 1], table64)
